# baseline (speedup 1.0000x reference)
import functools

import jax
import jax.numpy as jnp
from jax import lax
from jax.experimental import pallas as pl
from jax.experimental.pallas import tpu as pltpu
from jax.experimental.pallas import tpu_sc as plsc

N = 10000
E = 640000
E2 = 65536
D_EDGE = 768

NPAD = 10240
CH = 128
NW = 32
EPW = 20480
EPAD = NW * EPW
NCHUNK = EPW // CH
E2PW = E2 // NW
NCHUNK2 = E2PW // CH

_MESH = plsc.VectorSubcoreMesh(core_axis_name="c", subcore_axis_name="s")


def _wid():
    return lax.axis_index("s") * 2 + lax.axis_index("c")


NB = 4
NGRP = NCHUNK // NB


def _sc_pass1_body(src2d_hbm, dst2d_hbm, xpad_hbm, ones_hbm, zeros1_hbm,
                   s_out, deg_out,
                   sidx_all, didx_all, xv0, xv1, xv2, xv3, ones_v,
                   x_sh, s_sh, deg_sh,
                   g0, g1, g2, g3, sem_s):
    c = lax.axis_index("c")
    sid = lax.axis_index("s")
    wid = _wid()
    xvs = (xv0, xv1, xv2, xv3)
    gsems = (g0, g1, g2, g3)

    @pl.when(sid == 0)
    def _init():
        pltpu.sync_copy(zeros1_hbm, s_sh)
        pltpu.sync_copy(zeros1_hbm, deg_sh)
        pltpu.sync_copy(xpad_hbm, x_sh)

    pltpu.sync_copy(ones_hbm, ones_v)
    row0 = wid * NCHUNK
    pltpu.sync_copy(src2d_hbm.at[pl.ds(row0, NCHUNK)], sidx_all)
    pltpu.sync_copy(dst2d_hbm.at[pl.ds(row0, NCHUNK)], didx_all)
    plsc.subcore_barrier()

    def group(gi, carry):
        ci = gi * NB
        descs = []
        for b in range(NB):
            descs.append(pltpu.async_copy(
                x_sh.at[sidx_all.at[ci + b]], xvs[b], gsems[b]))
        sdescs = []
        for b in range(NB):
            descs[b].wait()
            sdescs.append(pltpu.async_copy(
                xvs[b], s_sh.at[didx_all.at[ci + b]], sem_s, add=True))
            sdescs.append(pltpu.async_copy(
                ones_v, deg_sh.at[didx_all.at[ci + b]], sem_s, add=True))
        for d in sdescs:
            d.wait()
        return carry

    lax.fori_loop(0, NGRP, group, 0)
    plsc.subcore_barrier()

    @pl.when(sid == 0)
    def _dump():
        pltpu.sync_copy(s_sh, s_out.at[c])
        pltpu.sync_copy(deg_sh, deg_out.at[c])


def _sc_pass1(src2d, dst2d, xpad, ones_v, zeros1):
    f = pl.kernel(
        _sc_pass1_body,
        out_type=(jax.ShapeDtypeStruct((2, NPAD), jnp.float32),
                  jax.ShapeDtypeStruct((2, NPAD), jnp.float32)),
        mesh=_MESH,
        scratch_types=[
            pltpu.VMEM((NCHUNK, CH), jnp.int32),
            pltpu.VMEM((NCHUNK, CH), jnp.int32),
            pltpu.VMEM((CH,), jnp.float32),
            pltpu.VMEM((CH,), jnp.float32),
            pltpu.VMEM((CH,), jnp.float32),
            pltpu.VMEM((CH,), jnp.float32),
            pltpu.VMEM((CH,), jnp.float32),
            pltpu.VMEM_SHARED((NPAD,), jnp.float32),
            pltpu.VMEM_SHARED((NPAD,), jnp.float32),
            pltpu.VMEM_SHARED((NPAD,), jnp.float32),
            pltpu.SemaphoreType.DMA,
            pltpu.SemaphoreType.DMA,
            pltpu.SemaphoreType.DMA,
            pltpu.SemaphoreType.DMA,
            pltpu.SemaphoreType.DMA,
        ],
    )
    return f(src2d, dst2d, xpad, ones_v, zeros1)


IB = 16
NIB = NCHUNK // IB


NB2 = 2


def _sc_pass2_body(src2d_hbm, dst2d_hbm, gpad_hbm, zeros128_hbm,
                   a2_out,
                   sidx_blk, didx_blk, r0, r1,
                   a2_sh,
                   g0, g1, sem_s):
    c = lax.axis_index("c")
    sid = lax.axis_index("s")
    wid = _wid()
    rows = (r0, r1)
    gsems = (g0, g1)

    @pl.when(sid == 0)
    def _init():
        pltpu.sync_copy(zeros128_hbm, a2_sh)

    row0 = wid * NCHUNK
    plsc.subcore_barrier()

    def block(bi, carry):
        brow = pl.multiple_of(row0 + bi * IB, 8)
        pltpu.sync_copy(src2d_hbm.at[pl.ds(brow, IB)], sidx_blk)
        pltpu.sync_copy(dst2d_hbm.at[pl.ds(brow, IB)], didx_blk)

        def group(gi, carry2):
            ci = gi * NB2
            descs = []
            for b in range(NB2):
                descs.append(pltpu.async_copy(
                    gpad_hbm.at[sidx_blk.at[ci + b]], rows[b], gsems[b]))
            sdescs = []
            for b in range(NB2):
                descs[b].wait()
                sdescs.append(pltpu.async_copy(
                    rows[b], a2_sh.at[didx_blk.at[ci + b]], sem_s, add=True))
            for d in sdescs:
                d.wait()
            return carry2

        lax.fori_loop(0, IB // NB2, group, 0)
        return carry

    lax.fori_loop(0, NIB, block, 0)
    plsc.subcore_barrier()

    @pl.when(sid == 0)
    def _dump():
        pltpu.sync_copy(a2_sh, a2_out.at[c])


def _sc_pass2(src2d, dst2d, gpad, zeros128):
    f = pl.kernel(
        _sc_pass2_body,
        out_type=jax.ShapeDtypeStruct((2, NPAD, 128), jnp.float32),
        mesh=_MESH,
        scratch_types=[
            pltpu.VMEM((IB, CH), jnp.int32),
            pltpu.VMEM((IB, CH), jnp.int32),
            pltpu.VMEM((CH, 128), jnp.float32),
            pltpu.VMEM((CH, 128), jnp.float32),
            pltpu.VMEM_SHARED((NPAD, 128), jnp.float32),
            pltpu.SemaphoreType.DMA,
            pltpu.SemaphoreType.DMA,
            pltpu.SemaphoreType.DMA,
        ],
    )
    return f(src2d, dst2d, gpad, zeros128)


def _sc_pass3_body(ei0_2d_hbm, ei1_2d_hbm, z_hbm, u0_out, u1_out,
                   aidx_all, bidx_all, za0, za1, zb0, zb1,
                   g0, g1, g2, g3, sem_w):
    wid = _wid()
    zas = (za0, za1)
    zbs = (zb0, zb1)
    row0 = wid * NCHUNK2
    pltpu.sync_copy(ei0_2d_hbm.at[pl.ds(row0, NCHUNK2)], aidx_all)
    pltpu.sync_copy(ei1_2d_hbm.at[pl.ds(row0, NCHUNK2)], bidx_all)

    def group(gi, carry):
        ci = gi * 2
        descs = []
        for b in range(2):
            descs.append(pltpu.async_copy(
                z_hbm.at[aidx_all.at[ci + b]], zas[b], (g0, g1)[b]))
            descs.append(pltpu.async_copy(
                z_hbm.at[bidx_all.at[ci + b]], zbs[b], (g2, g3)[b]))
        wdescs = []
        for b in range(2):
            e0 = pl.multiple_of((row0 + ci + b) * CH, 8)
            descs[2 * b].wait()
            wdescs.append(pltpu.async_copy(
                zas[b], u0_out.at[pl.ds(e0, CH)], sem_w))
            descs[2 * b + 1].wait()
            wdescs.append(pltpu.async_copy(
                zbs[b], u1_out.at[pl.ds(e0, CH)], sem_w))
        for d in wdescs:
            d.wait()
        return carry

    lax.fori_loop(0, NCHUNK2 // 2, group, 0)


def _sc_pass3(ei0_2d, ei1_2d, zpad):
    f = pl.kernel(
        _sc_pass3_body,
        out_type=(jax.ShapeDtypeStruct((E2, 128), jnp.float32),
                  jax.ShapeDtypeStruct((E2, 128), jnp.float32)),
        mesh=_MESH,
        scratch_types=[
            pltpu.VMEM((NCHUNK2, CH), jnp.int32),
            pltpu.VMEM((NCHUNK2, CH), jnp.int32),
            pltpu.VMEM((CH, 128), jnp.float32),
            pltpu.VMEM((CH, 128), jnp.float32),
            pltpu.VMEM((CH, 128), jnp.float32),
            pltpu.VMEM((CH, 128), jnp.float32),
            pltpu.SemaphoreType.DMA,
            pltpu.SemaphoreType.DMA,
            pltpu.SemaphoreType.DMA,
            pltpu.SemaphoreType.DMA,
            pltpu.SemaphoreType.DMA,
        ],
    )
    return f(ei0_2d, ei1_2d, zpad)


def _tc1_body(x_ref, scols_ref, degcols_ref, w1_ref, r1_ref, b1_ref,
              w2_ref, r2_ref, g_ref, hr_ref):
    s = scols_ref[:, 0:1] + scols_ref[:, 1:2]
    deg = jnp.maximum(degcols_ref[:, 0:1] + degcols_ref[:, 1:2], 1.0)
    sbar = s / deg
    h = jnp.maximum(sbar * w1_ref[...] + x_ref[...] * r1_ref[...] + b1_ref[...], 0.0)
    g_ref[...] = jnp.dot(h, w2_ref[...], preferred_element_type=jnp.float32)
    hr_ref[...] = jnp.dot(h, r2_ref[...], preferred_element_type=jnp.float32)


def _tc_stage1(x, scols, degcols, W1, root1, b1, W2pad, root2):
    return pl.pallas_call(
        _tc1_body,
        out_shape=(jax.ShapeDtypeStruct((N, 128), jnp.float32),
                   jax.ShapeDtypeStruct((N, 64), jnp.float32)),
    )(x, scols, degcols, W1, root1, b1, W2pad, root2)


def _tc2_body(a2_ref, degcols_ref, hr_ref, b2_ref, z_ref):
    deg = jnp.maximum(degcols_ref[:, 0:1] + degcols_ref[:, 1:2], 1.0)
    mean = (a2_ref[0, :N, :64] + a2_ref[1, :N, :64]) / deg
    z = jnp.maximum(mean + hr_ref[...] + b2_ref[...], 0.0)
    z_ref[...] = jnp.concatenate([z, jnp.zeros_like(z)], axis=1)


def _tc_stage2(a2, degcols, hr, b2):
    return pl.pallas_call(
        _tc2_body,
        out_shape=jax.ShapeDtypeStruct((N, 128), jnp.float32),
    )(a2, degcols, hr, b2)


def _tcp_body(ea_ref, w1b_ref, bl1_ref, p_ref):
    p_ref[...] = jnp.dot(ea_ref[...], w1b_ref[...],
                         preferred_element_type=jnp.float32) + bl1_ref[...]


def _tc_edge_proj(ea, Wl1b, bl1):
    R = 2048
    return pl.pallas_call(
        _tcp_body,
        grid=(E2 // R,),
        in_specs=[
            pl.BlockSpec((R, D_EDGE), lambda i: (i, 0)),
            pl.BlockSpec((D_EDGE, 128), lambda i: (0, 0)),
            pl.BlockSpec((1, 128), lambda i: (0, 0)),
        ],
        out_specs=pl.BlockSpec((R, 128), lambda i: (i, 0)),
        out_shape=jax.ShapeDtypeStruct((E2, 128), jnp.float32),
    )(ea, Wl1b, bl1)


def _tc3_body(u0_ref, u1_ref, p_ref, w1a_ref, wl2_ref, bl2_ref, out_ref):
    u = u0_ref[...] * u1_ref[...]
    hid = jnp.dot(u, w1a_ref[...], preferred_element_type=jnp.float32)
    hid = jnp.maximum(hid + p_ref[...], 0.0)
    lg = jnp.dot(hid, wl2_ref[...], preferred_element_type=jnp.float32) + bl2_ref[...]
    m = jnp.max(lg, axis=1, keepdims=True)
    p = jnp.exp(lg - m)
    out_ref[...] = p / jnp.sum(p, axis=1, keepdims=True)


def _tc_decode(u0, u1, pmat, Wl1a, Wl2, bl2):
    R = 2048
    return pl.pallas_call(
        _tc3_body,
        grid=(E2 // R,),
        in_specs=[
            pl.BlockSpec((R, 128), lambda i: (i, 0)),
            pl.BlockSpec((R, 128), lambda i: (i, 0)),
            pl.BlockSpec((R, 128), lambda i: (i, 0)),
            pl.BlockSpec((128, 128), lambda i: (0, 0)),
            pl.BlockSpec((128, 5), lambda i: (0, 0)),
            pl.BlockSpec((1, 5), lambda i: (0, 0)),
        ],
        out_specs=pl.BlockSpec((R, 5), lambda i: (i, 0)),
        out_shape=jax.ShapeDtypeStruct((E2, 5), jnp.float32),
    )(u0, u1, pmat, Wl1a, Wl2, bl2)


def kernel(x, train_edge_index, edge_index, edge_attr,
           W1, root1, b1, W2, root2, b2, Wl1, bl1, Wl2, bl2):
    f32 = jnp.float32

    pad = N + (jnp.arange(EPAD - E, dtype=jnp.int32) % (NPAD - N))
    src = jnp.concatenate([train_edge_index[0], pad]).reshape(EPAD // CH, CH)
    dst = jnp.concatenate([train_edge_index[1], pad]).reshape(EPAD // CH, CH)

    xpad = jnp.zeros((NPAD,), f32).at[:N].set(x[:, 0])
    ones_v = jnp.ones((CH,), f32)
    zeros1 = jnp.zeros((NPAD,), f32)
    zeros128 = jnp.zeros((NPAD, 128), f32)

    pmat = _tc_edge_proj(edge_attr, Wl1[64:], bl1.reshape(1, 128))

    s2, deg2 = _sc_pass1(src, dst, xpad, ones_v, zeros1)
    scols = s2[:, :N].T
    degcols = deg2[:, :N].T

    W2pad = jnp.concatenate([W2, jnp.zeros((128, 64), f32)], axis=1)
    g, hr = _tc_stage1(x, scols, degcols, W1, root1, b1.reshape(1, 128),
                       W2pad, root2)
    gpad = jnp.zeros((NPAD, 128), f32).at[:N].set(g)

    a2 = _sc_pass2(src, dst, gpad, zeros128)
    z = _tc_stage2(a2, degcols, hr, b2.reshape(1, 64))

    zpad = jnp.zeros((NPAD, 128), f32).at[:N].set(z)
    u0, u1 = _sc_pass3(edge_index[0].reshape(E2 // CH, CH),
                       edge_index[1].reshape(E2 // CH, CH), zpad)

    Wl1a = jnp.concatenate([Wl1[:64], jnp.zeros((64, 128), f32)], axis=0)
    return _tc_decode(u0, u1, pmat, Wl1a, Wl2, bl2.reshape(1, 5))


# speedup vs baseline: 1.0306x; 1.0306x over previous
import functools

import jax
import jax.numpy as jnp
from jax import lax
from jax.experimental import pallas as pl
from jax.experimental.pallas import tpu as pltpu
from jax.experimental.pallas import tpu_sc as plsc

N = 10000
E = 640000
E2 = 65536
D_EDGE = 768

NPAD = 10240
CH = 128
NW = 32
EPW = 20480
EPAD = NW * EPW
NCHUNK = EPW // CH
E2PW = E2 // NW
NCHUNK2 = E2PW // CH

_MESH = plsc.VectorSubcoreMesh(core_axis_name="c", subcore_axis_name="s")


def _wid():
    return lax.axis_index("s") * 2 + lax.axis_index("c")


NB = 4
NGRP = NCHUNK // NB


def _sc_pass1_body(src2d_hbm, dst2d_hbm, xpad_hbm, ones_hbm, zeros1_hbm,
                   s_out, deg_out,
                   sidx_all, didx_all, xv0, xv1, xv2, xv3, ones_v,
                   x_sh, s_sh, deg_sh,
                   g0, g1, g2, g3, sem_s):
    c = lax.axis_index("c")
    sid = lax.axis_index("s")
    wid = _wid()
    xvs = (xv0, xv1, xv2, xv3)
    gsems = (g0, g1, g2, g3)

    @pl.when(sid == 0)
    def _init():
        pltpu.sync_copy(zeros1_hbm, s_sh)
        pltpu.sync_copy(zeros1_hbm, deg_sh)
        pltpu.sync_copy(xpad_hbm, x_sh)

    pltpu.sync_copy(ones_hbm, ones_v)
    row0 = wid * NCHUNK
    pltpu.sync_copy(src2d_hbm.at[pl.ds(row0, NCHUNK)], sidx_all)
    pltpu.sync_copy(dst2d_hbm.at[pl.ds(row0, NCHUNK)], didx_all)
    plsc.subcore_barrier()

    def group(gi, carry):
        ci = gi * NB
        descs = []
        for b in range(NB):
            descs.append(pltpu.async_copy(
                x_sh.at[sidx_all.at[ci + b]], xvs[b], gsems[b]))
        sdescs = []
        for b in range(NB):
            descs[b].wait()
            sdescs.append(pltpu.async_copy(
                xvs[b], s_sh.at[didx_all.at[ci + b]], sem_s, add=True))
            sdescs.append(pltpu.async_copy(
                ones_v, deg_sh.at[didx_all.at[ci + b]], sem_s, add=True))
        for d in sdescs:
            d.wait()
        return carry

    lax.fori_loop(0, NGRP, group, 0)
    plsc.subcore_barrier()

    @pl.when(sid == 0)
    def _dump():
        pltpu.sync_copy(s_sh, s_out.at[c])
        pltpu.sync_copy(deg_sh, deg_out.at[c])


def _sc_pass1(src2d, dst2d, xpad, ones_v, zeros1):
    f = pl.kernel(
        _sc_pass1_body,
        out_type=(jax.ShapeDtypeStruct((2, NPAD), jnp.float32),
                  jax.ShapeDtypeStruct((2, NPAD), jnp.float32)),
        mesh=_MESH,
        scratch_types=[
            pltpu.VMEM((NCHUNK, CH), jnp.int32),
            pltpu.VMEM((NCHUNK, CH), jnp.int32),
            pltpu.VMEM((CH,), jnp.float32),
            pltpu.VMEM((CH,), jnp.float32),
            pltpu.VMEM((CH,), jnp.float32),
            pltpu.VMEM((CH,), jnp.float32),
            pltpu.VMEM((CH,), jnp.float32),
            pltpu.VMEM_SHARED((NPAD,), jnp.float32),
            pltpu.VMEM_SHARED((NPAD,), jnp.float32),
            pltpu.VMEM_SHARED((NPAD,), jnp.float32),
            pltpu.SemaphoreType.DMA,
            pltpu.SemaphoreType.DMA,
            pltpu.SemaphoreType.DMA,
            pltpu.SemaphoreType.DMA,
            pltpu.SemaphoreType.DMA,
        ],
    )
    return f(src2d, dst2d, xpad, ones_v, zeros1)


IB = 16
NIB = NCHUNK // IB


NB2 = 2


def _sc_pass2_body(src2d_hbm, dst2d_hbm, gpad_hbm, zeros128_hbm,
                   a2_out,
                   sidx_blk, didx_blk, r0, r1,
                   a2_sh,
                   g0, g1, sem_s):
    c = lax.axis_index("c")
    sid = lax.axis_index("s")
    wid = _wid()
    rows = (r0, r1)
    gsems = (g0, g1)

    @pl.when(sid == 0)
    def _init():
        pltpu.sync_copy(zeros128_hbm, a2_sh)

    row0 = wid * NCHUNK
    plsc.subcore_barrier()

    def block(bi, carry):
        brow = pl.multiple_of(row0 + bi * IB, 8)
        pltpu.sync_copy(src2d_hbm.at[pl.ds(brow, IB)], sidx_blk)
        pltpu.sync_copy(dst2d_hbm.at[pl.ds(brow, IB)], didx_blk)

        def group(gi, carry2):
            ci = gi * NB2
            descs = []
            for b in range(NB2):
                descs.append(pltpu.async_copy(
                    gpad_hbm.at[sidx_blk.at[ci + b]], rows[b], gsems[b]))
            sdescs = []
            for b in range(NB2):
                descs[b].wait()
                sdescs.append(pltpu.async_copy(
                    rows[b], a2_sh.at[didx_blk.at[ci + b]], sem_s, add=True))
            for d in sdescs:
                d.wait()
            return carry2

        lax.fori_loop(0, IB // NB2, group, 0)
        return carry

    lax.fori_loop(0, NIB, block, 0)
    plsc.subcore_barrier()

    @pl.when(sid == 0)
    def _dump():
        pltpu.sync_copy(a2_sh, a2_out.at[c])


def _sc_pass2(src2d, dst2d, gpad, zeros128):
    f = pl.kernel(
        _sc_pass2_body,
        out_type=jax.ShapeDtypeStruct((2, NPAD, 128), jnp.float32),
        mesh=_MESH,
        scratch_types=[
            pltpu.VMEM((IB, CH), jnp.int32),
            pltpu.VMEM((IB, CH), jnp.int32),
            pltpu.VMEM((CH, 128), jnp.float32),
            pltpu.VMEM((CH, 128), jnp.float32),
            pltpu.VMEM_SHARED((NPAD, 128), jnp.float32),
            pltpu.SemaphoreType.DMA,
            pltpu.SemaphoreType.DMA,
            pltpu.SemaphoreType.DMA,
        ],
    )
    return f(src2d, dst2d, gpad, zeros128)


def _mul_rows(za, zb, ub):
    def row(r, carry):
        for k in range(8):
            sl = pl.ds(16 * k, 16)
            ub[r, sl] = za[r, sl] * zb[r, sl]
        return carry

    lax.fori_loop(0, CH, row, 0)


def _sc_pass3_body(ei0_2d_hbm, ei1_2d_hbm, z_hbm, u_out,
                   aidx_all, bidx_all, za0, za1, zb0, zb1, ub0, ub1,
                   g0, g1, g2, g3, sem_w):
    wid = _wid()
    zas = (za0, za1)
    zbs = (zb0, zb1)
    ubs = (ub0, ub1)
    row0 = wid * NCHUNK2
    pltpu.sync_copy(ei0_2d_hbm.at[pl.ds(row0, NCHUNK2)], aidx_all)
    pltpu.sync_copy(ei1_2d_hbm.at[pl.ds(row0, NCHUNK2)], bidx_all)

    def group(gi, carry):
        ci = gi * 2
        descs = []
        for b in range(2):
            descs.append(pltpu.async_copy(
                z_hbm.at[aidx_all.at[ci + b]], zas[b], (g0, g1)[b]))
            descs.append(pltpu.async_copy(
                z_hbm.at[bidx_all.at[ci + b]], zbs[b], (g2, g3)[b]))
        wdescs = []
        for b in range(2):
            e0 = pl.multiple_of((row0 + ci + b) * CH, 8)
            descs[2 * b].wait()
            descs[2 * b + 1].wait()
            _mul_rows(zas[b], zbs[b], ubs[b])
            wdescs.append(pltpu.async_copy(
                ubs[b], u_out.at[pl.ds(e0, CH)], sem_w))
        for d in wdescs:
            d.wait()
        return carry

    lax.fori_loop(0, NCHUNK2 // 2, group, 0)


def _sc_pass3(ei0_2d, ei1_2d, zpad):
    f = pl.kernel(
        _sc_pass3_body,
        out_type=jax.ShapeDtypeStruct((E2, 128), jnp.float32),
        mesh=_MESH,
        scratch_types=[
            pltpu.VMEM((NCHUNK2, CH), jnp.int32),
            pltpu.VMEM((NCHUNK2, CH), jnp.int32),
            pltpu.VMEM((CH, 128), jnp.float32),
            pltpu.VMEM((CH, 128), jnp.float32),
            pltpu.VMEM((CH, 128), jnp.float32),
            pltpu.VMEM((CH, 128), jnp.float32),
            pltpu.VMEM((CH, 128), jnp.float32),
            pltpu.VMEM((CH, 128), jnp.float32),
            pltpu.SemaphoreType.DMA,
            pltpu.SemaphoreType.DMA,
            pltpu.SemaphoreType.DMA,
            pltpu.SemaphoreType.DMA,
            pltpu.SemaphoreType.DMA,
        ],
    )
    return f(ei0_2d, ei1_2d, zpad)


def _tc1_body(x_ref, scols_ref, degcols_ref, w1_ref, r1_ref, b1_ref,
              w2_ref, r2_ref, g_ref, hr_ref):
    s = scols_ref[:, 0:1] + scols_ref[:, 1:2]
    deg = jnp.maximum(degcols_ref[:, 0:1] + degcols_ref[:, 1:2], 1.0)
    sbar = s / deg
    h = jnp.maximum(sbar * w1_ref[...] + x_ref[...] * r1_ref[...] + b1_ref[...], 0.0)
    g_ref[...] = jnp.dot(h, w2_ref[...], preferred_element_type=jnp.float32)
    hr_ref[...] = jnp.dot(h, r2_ref[...], preferred_element_type=jnp.float32)


def _tc_stage1(x, scols, degcols, W1, root1, b1, W2pad, root2):
    return pl.pallas_call(
        _tc1_body,
        out_shape=(jax.ShapeDtypeStruct((N, 128), jnp.float32),
                   jax.ShapeDtypeStruct((N, 64), jnp.float32)),
    )(x, scols, degcols, W1, root1, b1, W2pad, root2)


def _tc2_body(a2_ref, degcols_ref, hr_ref, b2_ref, z_ref):
    deg = jnp.maximum(degcols_ref[:, 0:1] + degcols_ref[:, 1:2], 1.0)
    mean = (a2_ref[0, :N, :64] + a2_ref[1, :N, :64]) / deg
    z = jnp.maximum(mean + hr_ref[...] + b2_ref[...], 0.0)
    z_ref[...] = jnp.concatenate([z, jnp.zeros_like(z)], axis=1)


def _tc_stage2(a2, degcols, hr, b2):
    return pl.pallas_call(
        _tc2_body,
        out_shape=jax.ShapeDtypeStruct((N, 128), jnp.float32),
    )(a2, degcols, hr, b2)


def _tcp_body(ea_ref, w1b_ref, bl1_ref, p_ref):
    p_ref[...] = jnp.dot(ea_ref[...], w1b_ref[...],
                         preferred_element_type=jnp.float32) + bl1_ref[...]


def _tc_edge_proj(ea, Wl1b, bl1):
    R = 2048
    return pl.pallas_call(
        _tcp_body,
        grid=(E2 // R,),
        in_specs=[
            pl.BlockSpec((R, D_EDGE), lambda i: (i, 0)),
            pl.BlockSpec((D_EDGE, 128), lambda i: (0, 0)),
            pl.BlockSpec((1, 128), lambda i: (0, 0)),
        ],
        out_specs=pl.BlockSpec((R, 128), lambda i: (i, 0)),
        out_shape=jax.ShapeDtypeStruct((E2, 128), jnp.float32),
    )(ea, Wl1b, bl1)


def _tc3_body(u_ref, p_ref, w1a_ref, wl2_ref, bl2_ref, out_ref):
    hid = jnp.dot(u_ref[...], w1a_ref[...], preferred_element_type=jnp.float32)
    hid = jnp.maximum(hid + p_ref[...], 0.0)
    lg = jnp.dot(hid, wl2_ref[...], preferred_element_type=jnp.float32) + bl2_ref[...]
    m = jnp.max(lg, axis=1, keepdims=True)
    p = jnp.exp(lg - m)
    out_ref[...] = p / jnp.sum(p, axis=1, keepdims=True)


def _tc_decode(u, pmat, Wl1a, Wl2, bl2):
    R = 2048
    return pl.pallas_call(
        _tc3_body,
        grid=(E2 // R,),
        in_specs=[
            pl.BlockSpec((R, 128), lambda i: (i, 0)),
            pl.BlockSpec((R, 128), lambda i: (i, 0)),
            pl.BlockSpec((128, 128), lambda i: (0, 0)),
            pl.BlockSpec((128, 5), lambda i: (0, 0)),
            pl.BlockSpec((1, 5), lambda i: (0, 0)),
        ],
        out_specs=pl.BlockSpec((R, 5), lambda i: (i, 0)),
        out_shape=jax.ShapeDtypeStruct((E2, 5), jnp.float32),
    )(u, pmat, Wl1a, Wl2, bl2)


def kernel(x, train_edge_index, edge_index, edge_attr,
           W1, root1, b1, W2, root2, b2, Wl1, bl1, Wl2, bl2):
    f32 = jnp.float32

    pad = N + (jnp.arange(EPAD - E, dtype=jnp.int32) % (NPAD - N))
    src = jnp.concatenate([train_edge_index[0], pad]).reshape(EPAD // CH, CH)
    dst = jnp.concatenate([train_edge_index[1], pad]).reshape(EPAD // CH, CH)

    xpad = jnp.zeros((NPAD,), f32).at[:N].set(x[:, 0])
    ones_v = jnp.ones((CH,), f32)
    zeros1 = jnp.zeros((NPAD,), f32)
    zeros128 = jnp.zeros((NPAD, 128), f32)

    pmat = _tc_edge_proj(edge_attr, Wl1[64:], bl1.reshape(1, 128))

    s2, deg2 = _sc_pass1(src, dst, xpad, ones_v, zeros1)
    scols = s2[:, :N].T
    degcols = deg2[:, :N].T

    W2pad = jnp.concatenate([W2, jnp.zeros((128, 64), f32)], axis=1)
    g, hr = _tc_stage1(x, scols, degcols, W1, root1, b1.reshape(1, 128),
                       W2pad, root2)
    gpad = jnp.zeros((NPAD, 128), f32).at[:N].set(g)

    a2 = _sc_pass2(src, dst, gpad, zeros128)
    z = _tc_stage2(a2, degcols, hr, b2.reshape(1, 64))

    zpad = jnp.zeros((NPAD, 128), f32).at[:N].set(z)
    u = _sc_pass3(edge_index[0].reshape(E2 // CH, CH),
                  edge_index[1].reshape(E2 // CH, CH), zpad)

    Wl1a = jnp.concatenate([Wl1[:64], jnp.zeros((64, 128), f32)], axis=0)
    return _tc_decode(u, pmat, Wl1a, Wl2, bl2.reshape(1, 5))


# speedup vs baseline: 1.0453x; 1.0143x over previous
import functools

import jax
import jax.numpy as jnp
from jax import lax
from jax.experimental import pallas as pl
from jax.experimental.pallas import tpu as pltpu
from jax.experimental.pallas import tpu_sc as plsc

N = 10000
E = 640000
E2 = 65536
D_EDGE = 768

NPAD = 10240
CH = 128
NW = 32
EPW = 20480
EPAD = NW * EPW
NCHUNK = EPW // CH
E2PW = E2 // NW
NCHUNK2 = E2PW // CH

_MESH = plsc.VectorSubcoreMesh(core_axis_name="c", subcore_axis_name="s")


def _wid():
    return lax.axis_index("s") * 2 + lax.axis_index("c")


NB = 4
NGRP = NCHUNK // NB


def _sc_pass1_body(src2d_hbm, dst2d_hbm, xpad_hbm, ones_hbm, zeros1_hbm,
                   s_out, deg_out,
                   sidx_all, didx_all, xv0, xv1, xv2, xv3, ones_v,
                   x_sh, s_sh, deg_sh,
                   g0, g1, g2, g3, sem_s):
    c = lax.axis_index("c")
    sid = lax.axis_index("s")
    wid = _wid()
    xvs = (xv0, xv1, xv2, xv3)
    gsems = (g0, g1, g2, g3)

    @pl.when(sid == 0)
    def _init():
        pltpu.sync_copy(zeros1_hbm, s_sh)
        pltpu.sync_copy(zeros1_hbm, deg_sh)
        pltpu.sync_copy(xpad_hbm, x_sh)

    pltpu.sync_copy(ones_hbm, ones_v)
    row0 = wid * NCHUNK
    pltpu.sync_copy(src2d_hbm.at[pl.ds(row0, NCHUNK)], sidx_all)
    pltpu.sync_copy(dst2d_hbm.at[pl.ds(row0, NCHUNK)], didx_all)
    plsc.subcore_barrier()

    def group(gi, carry):
        ci = gi * NB
        descs = []
        for b in range(NB):
            descs.append(pltpu.async_copy(
                x_sh.at[sidx_all.at[ci + b]], xvs[b], gsems[b]))
        sdescs = []
        for b in range(NB):
            descs[b].wait()
            sdescs.append(pltpu.async_copy(
                xvs[b], s_sh.at[didx_all.at[ci + b]], sem_s, add=True))
            sdescs.append(pltpu.async_copy(
                ones_v, deg_sh.at[didx_all.at[ci + b]], sem_s, add=True))
        for d in sdescs:
            d.wait()
        return carry

    lax.fori_loop(0, NGRP, group, 0)
    plsc.subcore_barrier()

    @pl.when(sid == 0)
    def _dump():
        pltpu.sync_copy(s_sh, s_out.at[c])
        pltpu.sync_copy(deg_sh, deg_out.at[c])


def _sc_pass1(src2d, dst2d, xpad, ones_v, zeros1):
    f = pl.kernel(
        _sc_pass1_body,
        out_type=(jax.ShapeDtypeStruct((2, NPAD), jnp.float32),
                  jax.ShapeDtypeStruct((2, NPAD), jnp.float32)),
        mesh=_MESH,
        scratch_types=[
            pltpu.VMEM((NCHUNK, CH), jnp.int32),
            pltpu.VMEM((NCHUNK, CH), jnp.int32),
            pltpu.VMEM((CH,), jnp.float32),
            pltpu.VMEM((CH,), jnp.float32),
            pltpu.VMEM((CH,), jnp.float32),
            pltpu.VMEM((CH,), jnp.float32),
            pltpu.VMEM((CH,), jnp.float32),
            pltpu.VMEM_SHARED((NPAD,), jnp.float32),
            pltpu.VMEM_SHARED((NPAD,), jnp.float32),
            pltpu.VMEM_SHARED((NPAD,), jnp.float32),
            pltpu.SemaphoreType.DMA,
            pltpu.SemaphoreType.DMA,
            pltpu.SemaphoreType.DMA,
            pltpu.SemaphoreType.DMA,
            pltpu.SemaphoreType.DMA,
        ],
    )
    return f(src2d, dst2d, xpad, ones_v, zeros1)


IB = 16
NIB = NCHUNK // IB


NB2 = 2


def _sc_pass2_body(src2d_hbm, dst2d_hbm, gpad_hbm, zeros128_hbm,
                   a2_out,
                   sidx_blk, didx_blk, r0, r1,
                   a2_sh,
                   g0, g1, sem_s):
    c = lax.axis_index("c")
    sid = lax.axis_index("s")
    wid = _wid()
    rows = (r0, r1)
    gsems = (g0, g1)

    @pl.when(sid == 0)
    def _init():
        pltpu.sync_copy(zeros128_hbm, a2_sh)

    row0 = wid * NCHUNK
    plsc.subcore_barrier()

    def block(bi, carry):
        brow = pl.multiple_of(row0 + bi * IB, 8)
        pltpu.sync_copy(src2d_hbm.at[pl.ds(brow, IB)], sidx_blk)
        pltpu.sync_copy(dst2d_hbm.at[pl.ds(brow, IB)], didx_blk)

        def group(gi, carry2):
            ci = gi * NB2
            descs = []
            for b in range(NB2):
                descs.append(pltpu.async_copy(
                    gpad_hbm.at[sidx_blk.at[ci + b]], rows[b], gsems[b]))
            sdescs = []
            for b in range(NB2):
                descs[b].wait()
                sdescs.append(pltpu.async_copy(
                    rows[b], a2_sh.at[didx_blk.at[ci + b]], sem_s, add=True))
            for d in sdescs:
                d.wait()
            return carry2

        lax.fori_loop(0, IB // NB2, group, 0)
        return carry

    lax.fori_loop(0, NIB, block, 0)
    plsc.subcore_barrier()

    @pl.when(sid == 0)
    def _dump():
        pltpu.sync_copy(a2_sh, a2_out.at[c])


def _sc_pass2(src2d, dst2d, gpad, zeros128):
    f = pl.kernel(
        _sc_pass2_body,
        out_type=jax.ShapeDtypeStruct((2, NPAD, 128), jnp.float32),
        mesh=_MESH,
        scratch_types=[
            pltpu.VMEM((IB, CH), jnp.int32),
            pltpu.VMEM((IB, CH), jnp.int32),
            pltpu.VMEM((CH, 128), jnp.float32),
            pltpu.VMEM((CH, 128), jnp.float32),
            pltpu.VMEM_SHARED((NPAD, 128), jnp.float32),
            pltpu.SemaphoreType.DMA,
            pltpu.SemaphoreType.DMA,
            pltpu.SemaphoreType.DMA,
        ],
    )
    return f(src2d, dst2d, gpad, zeros128)


def _mul_rows(za, zb, ub):
    def row(r, carry):
        for k in range(8):
            sl = pl.ds(16 * k, 16)
            ub[r, sl] = za[r, sl] * zb[r, sl]
        return carry

    lax.fori_loop(0, CH, row, 0)


def _sc_pass3_body(ei0_2d_hbm, ei1_2d_hbm, z_hbm, u_out,
                   aidx_all, bidx_all, za0, za1, zb0, zb1, ub0, ub1,
                   g0, g1, g2, g3, sem_w):
    wid = _wid()
    zas = (za0, za1)
    zbs = (zb0, zb1)
    ubs = (ub0, ub1)
    row0 = wid * NCHUNK2
    pltpu.sync_copy(ei0_2d_hbm.at[pl.ds(row0, NCHUNK2)], aidx_all)
    pltpu.sync_copy(ei1_2d_hbm.at[pl.ds(row0, NCHUNK2)], bidx_all)

    def group(gi, carry):
        ci = gi * 2
        descs = []
        for b in range(2):
            descs.append(pltpu.async_copy(
                z_hbm.at[aidx_all.at[ci + b]], zas[b], (g0, g1)[b]))
            descs.append(pltpu.async_copy(
                z_hbm.at[bidx_all.at[ci + b]], zbs[b], (g2, g3)[b]))
        wdescs = []
        for b in range(2):
            e0 = pl.multiple_of((row0 + ci + b) * CH, 8)
            descs[2 * b].wait()
            descs[2 * b + 1].wait()
            _mul_rows(zas[b], zbs[b], ubs[b])
            wdescs.append(pltpu.async_copy(
                ubs[b], u_out.at[pl.ds(e0, CH)], sem_w))
        for d in wdescs:
            d.wait()
        return carry

    lax.fori_loop(0, NCHUNK2 // 2, group, 0)


def _sc_pass3(ei0_2d, ei1_2d, zpad):
    f = pl.kernel(
        _sc_pass3_body,
        out_type=jax.ShapeDtypeStruct((E2, 128), jnp.float32),
        mesh=_MESH,
        scratch_types=[
            pltpu.VMEM((NCHUNK2, CH), jnp.int32),
            pltpu.VMEM((NCHUNK2, CH), jnp.int32),
            pltpu.VMEM((CH, 128), jnp.float32),
            pltpu.VMEM((CH, 128), jnp.float32),
            pltpu.VMEM((CH, 128), jnp.float32),
            pltpu.VMEM((CH, 128), jnp.float32),
            pltpu.VMEM((CH, 128), jnp.float32),
            pltpu.VMEM((CH, 128), jnp.float32),
            pltpu.SemaphoreType.DMA,
            pltpu.SemaphoreType.DMA,
            pltpu.SemaphoreType.DMA,
            pltpu.SemaphoreType.DMA,
            pltpu.SemaphoreType.DMA,
        ],
    )
    return f(ei0_2d, ei1_2d, zpad)


def _tc1_body(x_ref, scols_ref, degcols_ref, w1_ref, r1_ref, b1_ref,
              w2_ref, r2_ref, g_ref, hr_ref):
    s = scols_ref[:, 0:1] + scols_ref[:, 1:2]
    deg = jnp.maximum(degcols_ref[:, 0:1] + degcols_ref[:, 1:2], 1.0)
    sbar = s / deg
    h = jnp.maximum(sbar * w1_ref[...] + x_ref[...] * r1_ref[...] + b1_ref[...], 0.0)
    g_ref[...] = jnp.dot(h, w2_ref[...], preferred_element_type=jnp.float32)
    hr_ref[...] = jnp.dot(h, r2_ref[...], preferred_element_type=jnp.float32)


def _tc_stage1(x, scols, degcols, W1, root1, b1, W2pad, root2):
    return pl.pallas_call(
        _tc1_body,
        out_shape=(jax.ShapeDtypeStruct((NPAD, 128), jnp.float32),
                   jax.ShapeDtypeStruct((NPAD, 64), jnp.float32)),
    )(x, scols, degcols, W1, root1, b1, W2pad, root2)


def _tc2_body(a2_ref, degcols_ref, hr_ref, b2_ref, z_ref):
    deg = jnp.maximum(degcols_ref[:, 0:1] + degcols_ref[:, 1:2], 1.0)
    mean = (a2_ref[0, :, :64] + a2_ref[1, :, :64]) / deg
    z = jnp.maximum(mean + hr_ref[...] + b2_ref[...], 0.0)
    z_ref[...] = jnp.concatenate([z, jnp.zeros_like(z)], axis=1)


def _tc_stage2(a2, degcols, hr, b2):
    return pl.pallas_call(
        _tc2_body,
        out_shape=jax.ShapeDtypeStruct((NPAD, 128), jnp.float32),
    )(a2, degcols, hr, b2)


def _tcp_body(ea_ref, w1b_ref, bl1_ref, p_ref):
    p_ref[...] = jnp.dot(ea_ref[...], w1b_ref[...],
                         preferred_element_type=jnp.float32) + bl1_ref[...]


def _tc_edge_proj(ea, Wl1b, bl1):
    R = 2048
    return pl.pallas_call(
        _tcp_body,
        grid=(E2 // R,),
        in_specs=[
            pl.BlockSpec((R, D_EDGE), lambda i: (i, 0)),
            pl.BlockSpec((D_EDGE, 128), lambda i: (0, 0)),
            pl.BlockSpec((1, 128), lambda i: (0, 0)),
        ],
        out_specs=pl.BlockSpec((R, 128), lambda i: (i, 0)),
        out_shape=jax.ShapeDtypeStruct((E2, 128), jnp.float32),
    )(ea, Wl1b, bl1)


def _tc3_body(u_ref, p_ref, w1a_ref, wl2_ref, bl2_ref, out_ref):
    hid = jnp.dot(u_ref[...], w1a_ref[...], preferred_element_type=jnp.float32)
    hid = jnp.maximum(hid + p_ref[...], 0.0)
    lg = jnp.dot(hid, wl2_ref[...], preferred_element_type=jnp.float32) + bl2_ref[...]
    m = jnp.max(lg, axis=1, keepdims=True)
    p = jnp.exp(lg - m)
    out_ref[...] = p / jnp.sum(p, axis=1, keepdims=True)


def _tc_decode(u, pmat, Wl1a, Wl2, bl2):
    R = 2048
    return pl.pallas_call(
        _tc3_body,
        grid=(E2 // R,),
        in_specs=[
            pl.BlockSpec((R, 128), lambda i: (i, 0)),
            pl.BlockSpec((R, 128), lambda i: (i, 0)),
            pl.BlockSpec((128, 128), lambda i: (0, 0)),
            pl.BlockSpec((128, 5), lambda i: (0, 0)),
            pl.BlockSpec((1, 5), lambda i: (0, 0)),
        ],
        out_specs=pl.BlockSpec((R, 5), lambda i: (i, 0)),
        out_shape=jax.ShapeDtypeStruct((E2, 5), jnp.float32),
    )(u, pmat, Wl1a, Wl2, bl2)


def kernel(x, train_edge_index, edge_index, edge_attr,
           W1, root1, b1, W2, root2, b2, Wl1, bl1, Wl2, bl2):
    f32 = jnp.float32

    pad = N + (jnp.arange(EPAD - E, dtype=jnp.int32) % (NPAD - N))
    src = jnp.concatenate([train_edge_index[0], pad]).reshape(EPAD // CH, CH)
    dst = jnp.concatenate([train_edge_index[1], pad]).reshape(EPAD // CH, CH)

    xpad = jnp.zeros((NPAD,), f32).at[:N].set(x[:, 0])
    ones_v = jnp.ones((CH,), f32)
    zeros1 = jnp.zeros((NPAD,), f32)
    zeros128 = jnp.zeros((NPAD, 128), f32)

    pmat = _tc_edge_proj(edge_attr, Wl1[64:], bl1.reshape(1, 128))

    s2, deg2 = _sc_pass1(src, dst, xpad, ones_v, zeros1)
    scols = s2.T
    degcols = deg2.T

    W2pad = jnp.concatenate([W2, jnp.zeros((128, 64), f32)], axis=1)
    gpad, hr = _tc_stage1(xpad.reshape(NPAD, 1), scols, degcols, W1, root1,
                          b1.reshape(1, 128), W2pad, root2)

    a2 = _sc_pass2(src, dst, gpad, zeros128)
    zpad = _tc_stage2(a2, degcols, hr, b2.reshape(1, 64))

    u = _sc_pass3(edge_index[0].reshape(E2 // CH, CH),
                  edge_index[1].reshape(E2 // CH, CH), zpad)

    Wl1a = jnp.concatenate([Wl1[:64], jnp.zeros((64, 128), f32)], axis=0)
    return _tc_decode(u, pmat, Wl1a, Wl2, bl2.reshape(1, 5))
